# baseline (speedup 1.0000x reference)
import functools

import jax
import jax.numpy as jnp
from jax import lax
from jax.experimental import pallas as pl
from jax.experimental.pallas import tpu as pltpu
from jax.experimental.pallas import tpu_sc as plsc

_INTERPRET = False


def _cdiv(a, b):
    return (a + b - 1) // b


def _lrelu(x):
    return jnp.where(x >= 0, x, 0.01 * x)


def _mm_nb(x, w, bm=1024):
    M, K = x.shape
    N = w.shape[1]

    def kern(x_ref, w_ref, o_ref):
        o_ref[...] = jnp.dot(x_ref[...], w_ref[...],
                             preferred_element_type=jnp.float32)

    return pl.pallas_call(
        kern,
        grid=(_cdiv(M, bm),),
        in_specs=[pl.BlockSpec((bm, K), lambda i: (i, 0)),
                  pl.BlockSpec((K, N), lambda i: (0, 0))],
        out_specs=pl.BlockSpec((bm, N), lambda i: (i, 0)),
        out_shape=jax.ShapeDtypeStruct((M, N), jnp.float32),
        interpret=_INTERPRET,
    )(x, w)


def _mm2(x1, w1, x2, w2, b, bm=1024):
    M, K1 = x1.shape
    K2 = x2.shape[1]
    N = w1.shape[1]

    def kern(x1_ref, w1_ref, x2_ref, w2_ref, b_ref, o_ref):
        acc = jnp.dot(x1_ref[...], w1_ref[...],
                      preferred_element_type=jnp.float32)
        acc = acc + jnp.dot(x2_ref[...], w2_ref[...],
                            preferred_element_type=jnp.float32)
        acc = acc + b_ref[...]
        o_ref[...] = jnp.where(acc >= 0, acc, 0.01 * acc)

    return pl.pallas_call(
        kern,
        grid=(_cdiv(M, bm),),
        in_specs=[pl.BlockSpec((bm, K1), lambda i: (i, 0)),
                  pl.BlockSpec((K1, N), lambda i: (0, 0)),
                  pl.BlockSpec((bm, K2), lambda i: (i, 0)),
                  pl.BlockSpec((K2, N), lambda i: (0, 0)),
                  pl.BlockSpec((1, N), lambda i: (0, 0))],
        out_specs=pl.BlockSpec((bm, N), lambda i: (i, 0)),
        out_shape=jax.ShapeDtypeStruct((M, N), jnp.float32),
        interpret=_INTERPRET,
    )(x1, w1, x2, w2, b.reshape(1, N))


def _mm_add(x, w, add, b, bm=1024):
    M, K = x.shape
    N = w.shape[1]

    def kern(x_ref, w_ref, a_ref, b_ref, o_ref):
        acc = jnp.dot(x_ref[...], w_ref[...],
                      preferred_element_type=jnp.float32)
        acc = acc + a_ref[...] + b_ref[...]
        o_ref[...] = jnp.where(acc >= 0, acc, 0.01 * acc)

    return pl.pallas_call(
        kern,
        grid=(_cdiv(M, bm),),
        in_specs=[pl.BlockSpec((bm, K), lambda i: (i, 0)),
                  pl.BlockSpec((K, N), lambda i: (0, 0)),
                  pl.BlockSpec((bm, N), lambda i: (i, 0)),
                  pl.BlockSpec((1, N), lambda i: (0, 0))],
        out_specs=pl.BlockSpec((bm, N), lambda i: (i, 0)),
        out_shape=jax.ShapeDtypeStruct((M, N), jnp.float32),
        interpret=_INTERPRET,
    )(x, w, add, b.reshape(1, N))


def _asm3(a, b, c, bias, bm=1024):
    M, N = a.shape

    def kern(a_ref, b_ref, c_ref, s_ref, o_ref):
        acc = a_ref[...] + b_ref[...] + c_ref[...] + s_ref[...]
        o_ref[...] = jnp.where(acc >= 0, acc, 0.01 * acc)

    return pl.pallas_call(
        kern,
        grid=(_cdiv(M, bm),),
        in_specs=[pl.BlockSpec((bm, N), lambda i: (i, 0)),
                  pl.BlockSpec((bm, N), lambda i: (i, 0)),
                  pl.BlockSpec((bm, N), lambda i: (i, 0)),
                  pl.BlockSpec((1, N), lambda i: (0, 0))],
        out_specs=pl.BlockSpec((bm, N), lambda i: (i, 0)),
        out_shape=jax.ShapeDtypeStruct((M, N), jnp.float32),
        interpret=_INTERPRET,
    )(a, b, c, bias.reshape(1, N))


def _add2(a, b, bm=1024):
    M, N = a.shape

    def kern(a_ref, b_ref, o_ref):
        acc = a_ref[...] + b_ref[...]
        o_ref[...] = jnp.where(acc >= 0, acc, 0.01 * acc)

    return pl.pallas_call(
        kern,
        grid=(_cdiv(M, bm),),
        in_specs=[pl.BlockSpec((bm, N), lambda i: (i, 0)),
                  pl.BlockSpec((bm, N), lambda i: (i, 0))],
        out_specs=pl.BlockSpec((bm, N), lambda i: (i, 0)),
        out_shape=jax.ShapeDtypeStruct((M, N), jnp.float32),
        interpret=_INTERPRET,
    )(a, b)


def _up_stage(z, w1, b1, w2, b2):
    LATD = z.shape[1]
    H = w1.shape[1]
    N = w2.shape[1]
    w1t = jnp.transpose(w1)
    b1t = b1.reshape(H, 1)
    w2t = jnp.transpose(w2)
    b2t = b2.reshape(N, 1)

    def kern(z_ref, w1_ref, b1_ref, w2_ref, b2_ref, o_ref):
        h = w1_ref[...] * z_ref[...] + b1_ref[...]
        h = jnp.where(h >= 0, h, 0.01 * h)
        o_ref[...] = jnp.dot(w2_ref[...], h,
                             preferred_element_type=jnp.float32) + b2_ref[...]

    return pl.pallas_call(
        kern,
        out_shape=jax.ShapeDtypeStruct((N, LATD), jnp.float32),
        interpret=_INTERPRET,
    )(z, w1t, b1t, w2t, b2t)


def _decoder(x, w1, b1, w2, b2, g, bt, bm=1024):
    M, K = x.shape
    H = w1.shape[1]
    F = w2.shape[1]

    def kern(x_ref, w1_ref, b1_ref, w2_ref, b2_ref, g_ref, bt_ref, o_ref):
        h = jnp.dot(x_ref[...], w1_ref[...],
                    preferred_element_type=jnp.float32) + b1_ref[...]
        h = jnp.where(h >= 0, h, 0.01 * h)
        y = jnp.dot(h, w2_ref[...],
                    preferred_element_type=jnp.float32) + b2_ref[...]
        mu = jnp.mean(y, axis=-1, keepdims=True)
        var = jnp.mean((y - mu) ** 2, axis=-1, keepdims=True)
        o_ref[...] = (y - mu) / jnp.sqrt(var + 1e-5) * g_ref[...] + bt_ref[...]

    return pl.pallas_call(
        kern,
        grid=(_cdiv(M, bm),),
        in_specs=[pl.BlockSpec((bm, K), lambda i: (i, 0)),
                  pl.BlockSpec((K, H), lambda i: (0, 0)),
                  pl.BlockSpec((1, H), lambda i: (0, 0)),
                  pl.BlockSpec((H, F), lambda i: (0, 0)),
                  pl.BlockSpec((1, F), lambda i: (0, 0)),
                  pl.BlockSpec((1, F), lambda i: (0, 0)),
                  pl.BlockSpec((1, F), lambda i: (0, 0))],
        out_specs=pl.BlockSpec((bm, F), lambda i: (i, 0)),
        out_shape=jax.ShapeDtypeStruct((M, F), jnp.float32),
        interpret=_INTERPRET,
    )(x, w1, b1.reshape(1, H), w2, b2.reshape(1, F),
      g.reshape(1, F), bt.reshape(1, F))


def _gather_rows(tbl, idx):
    return tbl[idx]


def _segsum(vals, seg, n):
    return jax.ops.segment_sum(vals, seg, num_segments=n)


def _padrow(a):
    return jnp.concatenate([a, jnp.zeros((8, a.shape[1]), a.dtype)], axis=0)


def _mpl_coarse(x, e, src, dst, p, n):
    din = x.shape[1]
    we = p['W_e']
    sp = _mm_nb(x, we[:din])
    dp = _mm_nb(x, we[din:2 * din])
    ep = _mm_nb(e, we[2 * din:])
    e_new = _asm3(_gather_rows(sp, src), _gather_rows(dp, dst), ep, p['b_e'])
    agg = _segsum(e_new, dst, n)
    wn = p['W_n']
    x_new = _mm2(x, wn[:din], agg, wn[din:], p['b_n'])
    return x_new, e_new


def _mpl_fine(x_c, e_c, src_t, dst_t, e_t, inv_full, dst_f, p, n_f):
    din = x_c.shape[1]
    we = p['W_e']
    sp = _padrow(_mm_nb(x_c, we[:din]))
    dp = _padrow(_mm_nb(x_c, we[din:2 * din]))
    ep = _padrow(_mm_nb(e_c, we[2 * din:]))
    e_new = _asm3(_gather_rows(sp, src_t), _gather_rows(dp, dst_t),
                  _gather_rows(ep, e_t), p['b_e'])
    agg = _segsum(e_new, dst_f, n_f)
    wn = p['W_n']
    xn1 = _gather_rows(_padrow(_mm_nb(x_c, wn[:din])), inv_full)
    x_new = _mm_add(agg, wn[din:], xn1, p['b_n'])
    return x_new, e_new


def _res_up(x, e, ei_c, ei_f, m_id, e_idx, n_c, n_f, e_f, rp):
    n_cc = x.shape[0]
    e_cc = e.shape[0]
    invm = jnp.full((n_f,), n_cc, jnp.int32).at[m_id].set(
        jnp.arange(n_cc, dtype=jnp.int32))
    invE = jnp.full((e_f,), e_cc, jnp.int32).at[e_idx].set(
        jnp.arange(e_cc, dtype=jnp.int32))
    src_f, dst_f = ei_f[0], ei_f[1]
    src_t = invm[src_f]
    dst_t = invm[dst_f]
    x_skip, _ = _mpl_fine(x, e, src_t, dst_t, invE, invm, dst_f,
                          rp['skip'], n_f)
    x1, e1 = _mpl_coarse(x, e, ei_c[0], ei_c[1], rp['mpl1'], n_c)
    x2, e2 = _mpl_fine(x1, e1, src_t, dst_t, invE, invm, dst_f,
                       rp['mpl2'], n_f)
    return _add2(x2, x_skip), e2


def kernel(z, edge_attr, params, edge_index2, edge_index1, edge_index0,
           m_id1, m_id0, e_idx1, e_idx0):
    p = params
    N2, N1, N0 = 2500, 5000, 10000
    E1, E0 = 80000, 160000
    ei2 = edge_index2.astype(jnp.int32)
    ei1 = edge_index1.astype(jnp.int32)
    ei0 = edge_index0.astype(jnp.int32)

    zr = z.reshape(1, -1)
    x = _up_stage(zr, p['up_W1'], p['up_b1'], p['up_W2'], p['up_b2'])
    e = edge_attr

    x, e = _mpl_coarse(x, e, ei2[0], ei2[1], p['bottom'], N2)
    x, e = _res_up(x, e, ei2, ei1, m_id1, e_idx1, N2, N1, E1, p['r0'])
    x, e = _res_up(x, e, ei1, ei0, m_id0, e_idx0, N1, N0, E0, p['r1'])
    x, e = _mpl_coarse(x, e, ei0[0], ei0[1], p['final'], N0)

    xn = _decoder(x, p['nd_W1'], p['nd_b1'], p['nd_W2'], p['nd_b2'],
                  p['nd_ln_g'], p['nd_ln_b'])
    en = _decoder(e, p['ed_W1'], p['ed_b1'], p['ed_W2'], p['ed_b2'],
                  p['ed_ln_g'], p['ed_ln_b'])
    return xn, en
